# baseline (speedup 1.0000x reference)
import jax
import jax.numpy as jnp
from jax import lax
from jax.experimental import pallas as pl
from jax.experimental.pallas import tpu as pltpu
from jax.experimental.pallas import tpu_sc as plsc

NUM_BUCKETS = 1000000
DIM = 64
B_ROWS = 4
SEQ = 8192
TOTAL = B_ROWS * SEQ

_info = plsc.get_sparse_core_info()
NC, NS, L = _info.num_cores, _info.num_subcores, _info.num_lanes
NW = NC * NS
B_PER_W = TOTAL // NW
N_VEC = B_PER_W // 16
GCHUNK = 128
N_G = B_PER_W // GCHUNK


def _sc_kernel(ids_hbm, table_hbm, out_hbm, ext_v, idx_v, rows_v, sem):
    wid = lax.axis_index("s") * NC + lax.axis_index("c")
    base = wid * B_PER_W

    pltpu.sync_copy(ids_hbm.at[pl.ds(base, B_PER_W + 8)], ext_v)

    lane = lax.iota(jnp.int32, 16)

    def hash_step(j, _):
        i0 = j * 16
        cur = ext_v[pl.ds(i0 + 8, 16)]
        prev = ext_v[pl.ds(i0 + 7, 16)]
        prev = prev * jnp.minimum((base + i0 + lane) % SEQ, 1)
        h = (prev * 31 + cur) % NUM_BUCKETS
        idx_v[pl.ds(i0, 16)] = h
        return 0

    lax.fori_loop(0, N_VEC, hash_step, 0, unroll=8)

    copies = []
    for g in range(N_G):
        copies.append(
            pltpu.async_copy(
                table_hbm.at[idx_v.at[pl.ds(g * GCHUNK, GCHUNK)]],
                rows_v.at[pl.ds(g * GCHUNK, GCHUNK)],
                sem,
            )
        )
    for c in copies:
        c.wait()

    pltpu.sync_copy(rows_v, out_hbm.at[pl.ds(base, B_PER_W)])


@jax.jit
def kernel(input_ids, emb_weight):
    ids_flat = input_ids.reshape(-1).astype(jnp.int32)
    ids_pad = jnp.concatenate([jnp.zeros((8,), jnp.int32), ids_flat])

    mesh = plsc.VectorSubcoreMesh(core_axis_name="c", subcore_axis_name="s")
    out = pl.kernel(
        _sc_kernel,
        mesh=mesh,
        out_type=jax.ShapeDtypeStruct((TOTAL, DIM), jnp.float32),
        scratch_types=[
            pltpu.VMEM((B_PER_W + 8,), jnp.int32),
            pltpu.VMEM((B_PER_W,), jnp.int32),
            pltpu.VMEM((B_PER_W, DIM), jnp.float32),
            pltpu.SemaphoreType.DMA,
        ],
        compiler_params=pltpu.CompilerParams(use_tc_tiling_on_sc=False),
    )(ids_pad, emb_weight)
    return out.reshape(B_ROWS, SEQ, DIM)
